# baseline (speedup 1.0000x reference)
import functools

import jax
import jax.numpy as jnp
from jax.experimental import pallas as pl
from jax.experimental.pallas import tpu as pltpu

B, T, D = 1, 2048, 768
FF = 3072
E = 8
K = 2

FFB = 768
NF = FF // FFB


def _router_kernel(x_ref, wg_ref, bg_ref, tw_ref, ti_ref):
    logits = jnp.dot(x_ref[...], wg_ref[...],
                     preferred_element_type=jnp.float32) + bg_ref[...]
    m = jnp.max(logits, axis=-1, keepdims=True)
    ex = jnp.exp(logits - m)
    probs = ex / jnp.sum(ex, axis=-1, keepdims=True)
    p1 = jnp.max(probs, axis=-1, keepdims=True)
    i1 = jnp.argmax(probs, axis=-1, keepdims=True)
    lane = jax.lax.broadcasted_iota(jnp.int32, probs.shape, 1)
    masked = jnp.where(lane == i1, -jnp.inf, probs)
    p2 = jnp.max(masked, axis=-1, keepdims=True)
    i2 = jnp.argmax(masked, axis=-1, keepdims=True)
    s = p1 + p2
    tw_ref[...] = jnp.concatenate([p1 / s, p2 / s], axis=1)
    ti_ref[...] = jnp.concatenate([i1, i2], axis=1).astype(jnp.int32)


def _moe_kernel(x_ref, w1_ref, b1_ref, w2_ref, b2_ref, ti_ref, tw_ref,
                out_ref):
    e = pl.program_id(0)
    f = pl.program_id(1)

    @pl.when((e == 0) & (f == 0))
    def _init():
        out_ref[...] = jnp.zeros_like(out_ref)

    h = jnp.dot(x_ref[...], w1_ref[0],
                preferred_element_type=jnp.float32) + b1_ref[0]
    h = jax.nn.gelu(h)
    p = jnp.dot(h, w2_ref[0], preferred_element_type=jnp.float32)

    ti = ti_ref[...]
    tw = tw_ref[...]
    gate = jnp.sum(jnp.where(ti == e, tw, 0.0), axis=1, keepdims=True)

    @pl.when(f == 0)
    def _bias2():
        out_ref[...] += gate * b2_ref[0]

    out_ref[...] += gate * p


def kernel(x, Wg, bg, W1, b1, W2, b2):
    xs = x.reshape(T, D)

    tw, ti = pl.pallas_call(
        _router_kernel,
        out_shape=(
            jax.ShapeDtypeStruct((T, K), jnp.float32),
            jax.ShapeDtypeStruct((T, K), jnp.int32),
        ),
    )(xs, Wg, bg.reshape(1, E))

    out = pl.pallas_call(
        _moe_kernel,
        grid=(E, NF),
        in_specs=[
            pl.BlockSpec((T, D), lambda e, f: (0, 0)),
            pl.BlockSpec((1, D, FFB), lambda e, f: (e, 0, f)),
            pl.BlockSpec((1, 1, FFB), lambda e, f: (e, 0, f)),
            pl.BlockSpec((1, FFB, D), lambda e, f: (e, f, 0)),
            pl.BlockSpec((1, 1, D), lambda e, f: (e, 0, 0)),
            pl.BlockSpec((T, K), lambda e, f: (0, 0)),
            pl.BlockSpec((T, K), lambda e, f: (0, 0)),
        ],
        out_specs=pl.BlockSpec((T, D), lambda e, f: (0, 0)),
        out_shape=jax.ShapeDtypeStruct((T, D), jnp.float32),
        compiler_params=pltpu.CompilerParams(
            dimension_semantics=("arbitrary", "arbitrary"),
        ),
    )(xs, W1, b1.reshape(E, 1, FF), W2, b2.reshape(E, 1, D), ti, tw)

    combined = out.reshape(B, T, D)
    aux_loss = jnp.zeros((), dtype=x.dtype)
    return combined, aux_loss, ti.reshape(B, T, K), tw.reshape(B, T, K)
